# baseline (speedup 1.0000x reference)
import functools

import jax
import jax.numpy as jnp
from jax import lax
from jax.experimental import pallas as pl
from jax.experimental.pallas import tpu as pltpu
from jax.experimental.pallas import tpu_sc as plsc

N = 10000
NP = 10112
E = 320000
B = 128
NSUB = 16
EPAD = ((E + NSUB * B - 1) // (NSUB * B)) * NSUB * B
NBLK = EPAD // (NSUB * B)
RPT = NP // 16
BR = 2528
F32 = jnp.float32


def _tc1_body(x_ref, w_ref, a_ref, pm_ref, hlo_ref, hhi_ref, sd_ref):
    h = jnp.dot(x_ref[...], w_ref[...], preferred_element_type=F32)
    hs = jnp.dot(h, pm_ref[...], preferred_element_type=F32)
    hlo_ref[...] = hs[:, :64].astype(jnp.bfloat16)
    hhi_ref[...] = hs[:, 64:].astype(jnp.bfloat16)
    sd_ref[...] = jnp.dot(h, a_ref[...], preferred_element_type=F32)


def _combine(num_ref, den_ref, hlo_ref, hhi_ref, sd_ref, b_ref,
             q_ref, p_ref, p16_ref, pmt_ref):
    z = jnp.dot(sd_ref[...], q_ref[...], preferred_element_type=F32)
    es = jnp.exp(jnp.maximum(z, 0.2 * z))
    esx = jnp.dot(es, p_ref[...], preferred_element_type=F32)
    denx = jnp.dot(den_ref[...], p16_ref[...],
                   preferred_element_type=F32) + esx
    hs = jnp.concatenate([hlo_ref[...], hhi_ref[...]], axis=1).astype(F32)
    hp = jnp.dot(hs, pmt_ref[...], preferred_element_type=F32)
    num = jnp.concatenate([num_ref[0], num_ref[1]], axis=1) + esx * hp
    return num / (denx + 1e-16) + b_ref[...]


def _tc2_body(num_ref, den_ref, hlo_ref, hhi_ref, sd_ref, b_ref, w_ref,
              a_ref, q_ref, p_ref, p16_ref, pmt_ref, pm_ref,
              h2lo_ref, h2hi_ref, sd2_ref):
    h1 = jnp.maximum(_combine(num_ref, den_ref, hlo_ref, hhi_ref, sd_ref,
                              b_ref, q_ref, p_ref, p16_ref, pmt_ref), 0.0)
    h2p = jnp.dot(h1, w_ref[...], preferred_element_type=F32)
    h2s = jnp.dot(h2p, pm_ref[...], preferred_element_type=F32)
    h2lo_ref[...] = h2s[:, :64].astype(jnp.bfloat16)
    h2hi_ref[...] = h2s[:, 64:].astype(jnp.bfloat16)
    sd2_ref[...] = jnp.dot(h2p, a_ref[...], preferred_element_type=F32)


def _tc3_body(num_ref, den_ref, hlo_ref, hhi_ref, sd_ref, b_ref,
              q_ref, p_ref, p16_ref, pmt_ref, out_ref):
    out_ref[...] = _combine(num_ref, den_ref, hlo_ref, hhi_ref, sd_ref,
                            b_ref, q_ref, p_ref, p16_ref, pmt_ref)


def _half_spec():
    return pl.BlockSpec((BR, 64), lambda i: (i, 0))


_BF16 = jnp.bfloat16


def _sd_spec():
    return pl.BlockSpec((BR, 16), lambda i: (i, 0))


def _const_spec(shape):
    return pl.BlockSpec(shape, lambda i: tuple(0 for _ in shape))


_tc1 = pl.pallas_call(
    _tc1_body,
    grid=(NP // BR,),
    in_specs=[pl.BlockSpec((BR, 128), lambda i: (i, 0)),
              _const_spec((128, 128)), _const_spec((128, 16)),
              _const_spec((128, 128))],
    out_specs=[_half_spec(), _half_spec(), _sd_spec()],
    out_shape=[jax.ShapeDtypeStruct((NP, 64), _BF16),
               jax.ShapeDtypeStruct((NP, 64), _BF16),
               jax.ShapeDtypeStruct((NP, 16), F32)],
)

_combine_specs = [pl.BlockSpec((2, BR, 64), lambda i: (0, i, 0)),
                  _sd_spec(),
                  _half_spec(), _half_spec(), _sd_spec(),
                  _const_spec((1, 128))]
_qpp_specs = [_const_spec((16, 8)), _const_spec((8, 128)),
              _const_spec((16, 128))]

_tc2 = pl.pallas_call(
    _tc2_body,
    grid=(NP // BR,),
    in_specs=_combine_specs + [_const_spec((128, 128)),
                               _const_spec((128, 16))] + _qpp_specs
             + [_const_spec((128, 128)), _const_spec((128, 128))],
    out_specs=[_half_spec(), _half_spec(), _sd_spec()],
    out_shape=[jax.ShapeDtypeStruct((NP, 64), _BF16),
               jax.ShapeDtypeStruct((NP, 64), _BF16),
               jax.ShapeDtypeStruct((NP, 16), F32)],
)

_tc3 = pl.pallas_call(
    _tc3_body,
    grid=(NP // BR,),
    in_specs=_combine_specs + _qpp_specs + [_const_spec((128, 128))],
    out_specs=pl.BlockSpec((BR, 128), lambda i: (i, 0)),
    out_shape=jax.ShapeDtypeStruct((NP, 128), F32),
)


_GATHER_DNUMS = lax.GatherDimensionNumbers(
    offset_dims=(), collapsed_slice_dims=(0,), start_index_map=(0,))


def _vgather(v, idx):
    return lax.gather(v, idx[:, None], _GATHER_DNUMS, slice_sizes=(1,),
                      mode=lax.GatherScatterMode.PROMISE_IN_BOUNDS)


def _sc_body(sd_hbm, h_hbm, esrc_hbm, edst_hbm, znum_hbm, zden_hbm,
             num_out, den_out,
             idx_s, idx_d, idx_h, sd_s, sd_d, h_rows, e16, msg,
             num_sp, den_sp, sem_i, sem_g, sem_s, sem_e):
    cid = lax.axis_index("c")
    sid = lax.axis_index("s")

    pltpu.sync_copy(znum_hbm, num_sp.at[pl.ds(sid * RPT, RPT)])
    pltpu.sync_copy(zden_hbm, den_sp.at[pl.ds(sid * RPT, RPT)])
    plsc.subcore_barrier()

    colr = (lax.iota(jnp.int32, 16) & 7) + 8
    e0 = sid * NBLK * B
    hoff = cid * NP

    def idx_copies(blk, p8):
        base = e0 + blk * B
        return (pltpu.make_async_copy(esrc_hbm.at[pl.ds(base, B)],
                                      idx_s.at[p8], sem_i.at[p8]),
                pltpu.make_async_copy(edst_hbm.at[pl.ds(base, B)],
                                      idx_d.at[p8], sem_i.at[p8]))

    def gather_copies(p8, g8):
        return (pltpu.make_async_copy(sd_hbm.at[idx_s.at[p8]],
                                      sd_s.at[g8], sem_g.at[g8]),
                pltpu.make_async_copy(sd_hbm.at[idx_d.at[p8]],
                                      sd_d.at[g8], sem_g.at[g8]),
                pltpu.make_async_copy(h_hbm.at[idx_h.at[p8]],
                                      h_rows.at[g8], sem_g.at[g8]))

    def msg_copy(p8, p2):
        return (pltpu.make_async_copy(msg.at[p2], num_sp.at[idx_d.at[p8]],
                                      sem_s.at[p2]),)

    def den_copy(p8, p2):
        return (pltpu.make_async_copy(e16.at[p2], den_sp.at[idx_d.at[p8]],
                                      sem_e.at[p2]),)

    def start(copies, add=False):
        for c in copies:
            c.start(add=add)

    def wait(copies):
        for c in copies:
            c.wait()

    def prep_gathers(blk, p8):
        wait(idx_copies(blk, p8))
        for j in range(B // 16):
            v = idx_s[p8, pl.ds(j * 16, 16)]
            idx_h[p8, pl.ds(j * 16, 16)] = v + hoff
        start(gather_copies(p8, lax.rem(blk, 6)))

    for blk in range(6):
        start(idx_copies(blk, blk))
    for blk in range(5):
        prep_gathers(blk, blk)

    def block_body(blk, carry):
        p2 = lax.rem(blk, 2)
        p4 = lax.rem(blk, 4)
        p8 = lax.rem(blk, 8)

        @pl.when(blk >= 2)
        def _():
            wait(msg_copy(lax.rem(blk + 6, 8), p2))

            @pl.when(cid == 0)
            def _():
                wait(den_copy(lax.rem(blk + 6, 8), p2))

        @pl.when(blk + 6 < NBLK)
        def _():
            start(idx_copies(blk + 6, lax.rem(blk + 6, 8)))

        g6 = lax.rem(blk, 6)
        wait(gather_copies(p8, g6))

        @pl.when(blk + 5 < NBLK)
        def _():
            prep_gathers(blk + 5, lax.rem(blk + 5, 8))

        hbase = cid * 4

        @plsc.parallel_loop(0, B, unroll=16)
        def _(b):
            vs = sd_s[g6, b, :]
            vd = sd_d[g6, b, :]
            vdr = _vgather(vd, colr)
            e = vs + vdr
            ex = jnp.exp(jnp.maximum(e, 0.2 * e))
            e16[p2, b, :] = ex
            for pair in range(2):
                w = h_rows[g6, b, pl.ds(pair * 16, 16)]
                va = lax.bitcast_convert_type(w << 16, F32)
                vb = lax.bitcast_convert_type(
                    w & jnp.int32(-65536), F32)
                g = hbase + pair * 2
                msg[p2, b, pl.ds(pair * 32, 16)] = (
                    va * _vgather(ex, jnp.full((16,), g, jnp.int32)))
                msg[p2, b, pl.ds(pair * 32 + 16, 16)] = (
                    vb * _vgather(ex, jnp.full((16,), g + 1, jnp.int32)))

        start(msg_copy(p8, p2), add=True)

        @pl.when(cid == 0)
        def _():
            start(den_copy(p8, p2), add=True)
        return carry

    lax.fori_loop(0, NBLK, block_body, 0)
    for tail in (NBLK - 2, NBLK - 1):
        wait(msg_copy(tail % 8, tail % 2))

        @pl.when(cid == 0)
        def _():
            wait(den_copy(tail % 4, tail % 2))
    plsc.subcore_barrier()

    pltpu.sync_copy(num_sp.at[pl.ds(sid * RPT, RPT)],
                    num_out.at[cid, pl.ds(sid * RPT, RPT), :])

    @pl.when(cid == 0)
    def _():
        pltpu.sync_copy(den_sp.at[pl.ds(sid * RPT, RPT)],
                        den_out.at[pl.ds(sid * RPT, RPT)])


_sc_edge = functools.partial(
    pl.kernel,
    mesh=plsc.VectorSubcoreMesh(core_axis_name="c", subcore_axis_name="s"),
    compiler_params=pltpu.CompilerParams(use_tc_tiling_on_sc=False),
    out_type=[jax.ShapeDtypeStruct((2, NP, 64), F32),
              jax.ShapeDtypeStruct((NP, 16), F32)],
    scratch_types=[
        pltpu.VMEM((8, B), jnp.int32),
        pltpu.VMEM((8, B), jnp.int32),
        pltpu.VMEM((8, B), jnp.int32),
        pltpu.VMEM((6, B, 16), F32),
        pltpu.VMEM((6, B, 16), F32),
        pltpu.VMEM((6, B, 32), jnp.int32),
        pltpu.VMEM((2, B, 16), F32),
        pltpu.VMEM((2, B, 64), F32),
        pltpu.VMEM_SHARED((NP, 64), F32),
        pltpu.VMEM_SHARED((NP, 16), F32),
        pltpu.SemaphoreType.DMA((8,)),
        pltpu.SemaphoreType.DMA((6,)),
        pltpu.SemaphoreType.DMA((2,)),
        pltpu.SemaphoreType.DMA((2,)),
    ],
)(_sc_body)


def _pack_rows(lo, hi):
    cat = jnp.concatenate([lo, hi], axis=0)
    return jax.lax.bitcast_convert_type(
        cat.reshape(2 * NP, 32, 2), jnp.int32)


def _expander(a):
    eye = jnp.eye(8, dtype=a.dtype)
    return (a[:, :, None] * eye[:, None, :]).reshape(128, 8)


def kernel(x, edge_index, W1, a_src1, a_dst1, b1, W2, a_src2, a_dst2, b2):
    ei = edge_index.astype(jnp.int32)
    pad = EPAD - E
    esrc = jnp.concatenate([ei[0], jnp.zeros((pad,), jnp.int32)])
    edst = jnp.concatenate([ei[1], jnp.full((pad,), N, jnp.int32)])
    x_pad = jnp.pad(x, ((0, NP - N), (0, 0)))

    A1 = jnp.concatenate([_expander(a_src1), _expander(a_dst1)], axis=1)
    A2 = jnp.concatenate([_expander(a_src2), _expander(a_dst2)], axis=1)
    eye8 = jnp.eye(8, dtype=F32)
    Q = jnp.concatenate([eye8, eye8], axis=0)
    P = jnp.repeat(eye8, 16, axis=1)
    P16 = jnp.concatenate([P, jnp.zeros((8, 128), F32)], axis=0)
    j = jnp.arange(128)
    p_half = j // 64
    w = j % 64
    perm = p_half * 64 + ((w // 32) * 2 + (w % 2)) * 16 + (w % 32) // 2
    PM = jnp.zeros((128, 128), F32).at[perm, j].set(1.0)
    PMT = PM.T
    znum = jnp.zeros((RPT, 64), F32)
    zden = jnp.zeros((RPT, 16), F32)
    b1r = b1.reshape(1, 128)
    b2r = b2.reshape(1, 128)

    h1lo, h1hi, sd1 = _tc1(x_pad, W1, A1, PM)
    h1cat = _pack_rows(h1lo, h1hi)
    num1, den1 = _sc_edge(sd1, h1cat, esrc, edst, znum, zden)
    h2lo, h2hi, sd2 = _tc2(num1, den1, h1lo, h1hi, sd1, b1r, W2, A2,
                           Q, P, P16, PMT, PM)
    h2cat = _pack_rows(h2lo, h2hi)
    num2, den2 = _sc_edge(sd2, h2cat, esrc, edst, znum, zden)
    out = _tc3(num2, den2, h2lo, h2hi, sd2, b2r, Q, P, P16, PMT)
    return out[:N]


# speedup vs baseline: 1.0349x; 1.0349x over previous
import functools

import numpy as np

import jax
import jax.numpy as jnp
from jax import lax
from jax.experimental import pallas as pl
from jax.experimental.pallas import tpu as pltpu
from jax.experimental.pallas import tpu_sc as plsc

N = 10000
NP = 10112
E = 320000
B = 128
NSUB = 16
EPAD = ((E + NSUB * B - 1) // (NSUB * B)) * NSUB * B
NBLK = EPAD // (NSUB * B)
RPT = NP // 16
BR = 2528
F32 = jnp.float32


def _tc1_body(x_ref, w_ref, a_ref, pm_ref, hlo_ref, hhi_ref, sd_ref):
    h = jnp.dot(x_ref[...], w_ref[...], preferred_element_type=F32)
    hs = jnp.dot(h, pm_ref[...], preferred_element_type=F32)
    hlo_ref[...] = hs[:, :64].astype(jnp.bfloat16)
    hhi_ref[...] = hs[:, 64:].astype(jnp.bfloat16)
    sd_ref[...] = jnp.dot(h, a_ref[...], preferred_element_type=F32)


def _combine(num_ref, den_ref, hlo_ref, hhi_ref, sd_ref, b_ref,
             q_ref, p_ref, p16_ref, pmt_ref):
    z = jnp.dot(sd_ref[...], q_ref[...], preferred_element_type=F32)
    es = jnp.exp(jnp.maximum(z, 0.2 * z))
    esx = jnp.dot(es, p_ref[...], preferred_element_type=F32)
    denx = jnp.dot(den_ref[...], p16_ref[...],
                   preferred_element_type=F32) + esx
    hs = jnp.concatenate([hlo_ref[...], hhi_ref[...]], axis=1).astype(F32)
    hp = jnp.dot(hs, pmt_ref[...], preferred_element_type=F32)
    num = jnp.concatenate([num_ref[0], num_ref[1]], axis=1) + esx * hp
    return num / (denx + 1e-16) + b_ref[...]


def _tc2_body(num_ref, den_ref, hlo_ref, hhi_ref, sd_ref, b_ref, w_ref,
              a_ref, q_ref, p_ref, p16_ref, pmt_ref, pm_ref,
              h2lo_ref, h2hi_ref, sd2_ref):
    h1 = jnp.maximum(_combine(num_ref, den_ref, hlo_ref, hhi_ref, sd_ref,
                              b_ref, q_ref, p_ref, p16_ref, pmt_ref), 0.0)
    h2p = jnp.dot(h1, w_ref[...], preferred_element_type=F32)
    h2s = jnp.dot(h2p, pm_ref[...], preferred_element_type=F32)
    h2lo_ref[...] = h2s[:, :64].astype(jnp.bfloat16)
    h2hi_ref[...] = h2s[:, 64:].astype(jnp.bfloat16)
    sd2_ref[...] = jnp.dot(h2p, a_ref[...], preferred_element_type=F32)


def _tc3_body(num_ref, den_ref, hlo_ref, hhi_ref, sd_ref, b_ref,
              q_ref, p_ref, p16_ref, pmt_ref, out_ref):
    out_ref[...] = _combine(num_ref, den_ref, hlo_ref, hhi_ref, sd_ref,
                            b_ref, q_ref, p_ref, p16_ref, pmt_ref)


def _half_spec():
    return pl.BlockSpec((BR, 64), lambda i: (i, 0))


_BF16 = jnp.bfloat16


def _sd_spec():
    return pl.BlockSpec((BR, 16), lambda i: (i, 0))


def _const_spec(shape):
    return pl.BlockSpec(shape, lambda i: tuple(0 for _ in shape))


_tc1 = pl.pallas_call(
    _tc1_body,
    grid=(NP // BR,),
    in_specs=[pl.BlockSpec((BR, 128), lambda i: (i, 0)),
              _const_spec((128, 128)), _const_spec((128, 16)),
              _const_spec((128, 128))],
    out_specs=[_half_spec(), _half_spec(), _sd_spec()],
    out_shape=[jax.ShapeDtypeStruct((NP, 64), _BF16),
               jax.ShapeDtypeStruct((NP, 64), _BF16),
               jax.ShapeDtypeStruct((NP, 16), F32)],
)

_combine_specs = [pl.BlockSpec((2, BR, 64), lambda i: (0, i, 0)),
                  _sd_spec(),
                  _half_spec(), _half_spec(), _sd_spec(),
                  _const_spec((1, 128))]
_qpp_specs = [_const_spec((16, 8)), _const_spec((8, 128)),
              _const_spec((16, 128))]

_tc2 = pl.pallas_call(
    _tc2_body,
    grid=(NP // BR,),
    in_specs=_combine_specs + [_const_spec((128, 128)),
                               _const_spec((128, 16))] + _qpp_specs
             + [_const_spec((128, 128)), _const_spec((128, 128))],
    out_specs=[_half_spec(), _half_spec(), _sd_spec()],
    out_shape=[jax.ShapeDtypeStruct((NP, 64), _BF16),
               jax.ShapeDtypeStruct((NP, 64), _BF16),
               jax.ShapeDtypeStruct((NP, 16), F32)],
)

BR3 = 2000
_combine_specs3 = [pl.BlockSpec((2, BR3, 64), lambda i: (0, i, 0)),
                   pl.BlockSpec((BR3, 16), lambda i: (i, 0)),
                   pl.BlockSpec((BR3, 64), lambda i: (i, 0)),
                   pl.BlockSpec((BR3, 64), lambda i: (i, 0)),
                   pl.BlockSpec((BR3, 16), lambda i: (i, 0)),
                   _const_spec((1, 128))]

_tc3 = pl.pallas_call(
    _tc3_body,
    grid=(N // BR3,),
    in_specs=_combine_specs3 + _qpp_specs + [_const_spec((128, 128))],
    out_specs=pl.BlockSpec((BR3, 128), lambda i: (i, 0)),
    out_shape=jax.ShapeDtypeStruct((N, 128), F32),
)


_GATHER_DNUMS = lax.GatherDimensionNumbers(
    offset_dims=(), collapsed_slice_dims=(0,), start_index_map=(0,))


def _vgather(v, idx):
    return lax.gather(v, idx[:, None], _GATHER_DNUMS, slice_sizes=(1,),
                      mode=lax.GatherScatterMode.PROMISE_IN_BOUNDS)


def _sc_body(sd_hbm, h_hbm, esrc_hbm, edst_hbm, znum_hbm, zden_hbm,
             num_out, den_out,
             idx_s, idx_d, idx_h, sd_s, sd_d, h_rows, e16, msg,
             num_sp, den_sp, sem_i, sem_g, sem_s, sem_e):
    cid = lax.axis_index("c")
    sid = lax.axis_index("s")

    pltpu.sync_copy(znum_hbm, num_sp.at[pl.ds(sid * RPT, RPT)])
    pltpu.sync_copy(zden_hbm, den_sp.at[pl.ds(sid * RPT, RPT)])
    plsc.subcore_barrier()

    colr = (lax.iota(jnp.int32, 16) & 7) + 8
    e0 = sid * NBLK * B
    hoff = cid * NP

    def idx_copies(blk, p8):
        base = e0 + blk * B
        return (pltpu.make_async_copy(esrc_hbm.at[pl.ds(base, B)],
                                      idx_s.at[p8], sem_i.at[p8]),
                pltpu.make_async_copy(edst_hbm.at[pl.ds(base, B)],
                                      idx_d.at[p8], sem_i.at[p8]))

    def gather_copies(p8, p4):
        return (pltpu.make_async_copy(sd_hbm.at[idx_s.at[p8]],
                                      sd_s.at[p4], sem_g.at[p4]),
                pltpu.make_async_copy(sd_hbm.at[idx_d.at[p8]],
                                      sd_d.at[p4], sem_g.at[p4]),
                pltpu.make_async_copy(h_hbm.at[idx_h.at[p8]],
                                      h_rows.at[p4], sem_g.at[p4]))

    def msg_copy(p8, p2):
        return (pltpu.make_async_copy(msg.at[p2], num_sp.at[idx_d.at[p8]],
                                      sem_s.at[p2]),)

    def den_copy(p8, p2):
        return (pltpu.make_async_copy(e16.at[p2], den_sp.at[idx_d.at[p8]],
                                      sem_e.at[p2]),)

    def start(copies, add=False):
        for c in copies:
            c.start(add=add)

    def wait(copies):
        for c in copies:
            c.wait()

    def prep_gathers(blk, p8):
        wait(idx_copies(blk, p8))
        for j in range(B // 16):
            v = idx_s[p8, pl.ds(j * 16, 16)]
            idx_h[p8, pl.ds(j * 16, 16)] = v + hoff
        start(gather_copies(p8, lax.rem(blk, 4)))

    for blk in range(4):
        start(idx_copies(blk, blk))
    for blk in range(3):
        prep_gathers(blk, blk)

    def block_body(blk, carry):
        p2 = lax.rem(blk, 2)
        p4 = lax.rem(blk, 4)
        p8 = lax.rem(blk, 8)

        @pl.when(blk >= 2)
        def _():
            wait(msg_copy(lax.rem(blk + 6, 8), p2))

            @pl.when(cid == 0)
            def _():
                wait(den_copy(lax.rem(blk + 6, 8), p2))

        @pl.when(blk + 4 < NBLK)
        def _():
            start(idx_copies(blk + 4, lax.rem(blk + 4, 8)))

        wait(gather_copies(p8, p4))

        @pl.when(blk + 3 < NBLK)
        def _():
            prep_gathers(blk + 3, lax.rem(blk + 3, 8))

        hbase = cid * 4

        @plsc.parallel_loop(0, B, unroll=16)
        def _(b):
            vs = sd_s[p4, b, :]
            vd = sd_d[p4, b, :]
            vdr = _vgather(vd, colr)
            e = vs + vdr
            ex = jnp.exp(jnp.maximum(e, 0.2 * e))
            e16[p2, b, :] = ex
            for pair in range(2):
                w = h_rows[p4, b, pl.ds(pair * 16, 16)]
                va = lax.bitcast_convert_type(w << 16, F32)
                vb = lax.bitcast_convert_type(
                    w & jnp.int32(-65536), F32)
                g = hbase + pair * 2
                msg[p2, b, pl.ds(pair * 32, 16)] = (
                    va * _vgather(ex, jnp.full((16,), g, jnp.int32)))
                msg[p2, b, pl.ds(pair * 32 + 16, 16)] = (
                    vb * _vgather(ex, jnp.full((16,), g + 1, jnp.int32)))

        start(msg_copy(p8, p2), add=True)

        @pl.when(cid == 0)
        def _():
            start(den_copy(p8, p2), add=True)
        return carry

    lax.fori_loop(0, NBLK, block_body, 0)
    for tail in (NBLK - 2, NBLK - 1):
        wait(msg_copy(tail % 8, tail % 2))

        @pl.when(cid == 0)
        def _():
            wait(den_copy(tail % 4, tail % 2))
    plsc.subcore_barrier()

    pltpu.sync_copy(num_sp.at[pl.ds(sid * RPT, RPT)],
                    num_out.at[cid, pl.ds(sid * RPT, RPT), :])

    @pl.when(cid == 0)
    def _():
        pltpu.sync_copy(den_sp.at[pl.ds(sid * RPT, RPT)],
                        den_out.at[pl.ds(sid * RPT, RPT)])


_sc_edge = functools.partial(
    pl.kernel,
    mesh=plsc.VectorSubcoreMesh(core_axis_name="c", subcore_axis_name="s"),
    compiler_params=pltpu.CompilerParams(use_tc_tiling_on_sc=False),
    out_type=[jax.ShapeDtypeStruct((2, NP, 64), F32),
              jax.ShapeDtypeStruct((NP, 16), F32)],
    scratch_types=[
        pltpu.VMEM((8, B), jnp.int32),
        pltpu.VMEM((8, B), jnp.int32),
        pltpu.VMEM((8, B), jnp.int32),
        pltpu.VMEM((4, B, 16), F32),
        pltpu.VMEM((4, B, 16), F32),
        pltpu.VMEM((4, B, 32), jnp.int32),
        pltpu.VMEM((2, B, 16), F32),
        pltpu.VMEM((2, B, 64), F32),
        pltpu.VMEM_SHARED((NP, 64), F32),
        pltpu.VMEM_SHARED((NP, 16), F32),
        pltpu.SemaphoreType.DMA((8,)),
        pltpu.SemaphoreType.DMA((4,)),
        pltpu.SemaphoreType.DMA((2,)),
        pltpu.SemaphoreType.DMA((2,)),
    ],
)(_sc_body)


_EYE8 = np.eye(8, dtype=np.float32)
_Q = np.concatenate([_EYE8, _EYE8], axis=0)
_P = np.repeat(_EYE8, 16, axis=1)
_P16 = np.concatenate([_P, np.zeros((8, 128), np.float32)], axis=0)
_j = np.arange(128)
_w = _j % 64
_perm = (_j // 64) * 64 + ((_w // 32) * 2 + (_w % 2)) * 16 + (_w % 32) // 2
_PM = np.zeros((128, 128), np.float32)
_PM[_perm, _j] = 1.0
_PMT = _PM.T.copy()
_ZNUM = np.zeros((RPT, 64), np.float32)
_ZDEN = np.zeros((RPT, 16), np.float32)


def _pack_rows(lo, hi):
    cat = jnp.concatenate([lo, hi], axis=0)
    return jax.lax.bitcast_convert_type(
        cat.reshape(2 * NP, 32, 2), jnp.int32)


def _expander(a):
    eye = jnp.eye(8, dtype=a.dtype)
    return (a[:, :, None] * eye[:, None, :]).reshape(128, 8)


def kernel(x, edge_index, W1, a_src1, a_dst1, b1, W2, a_src2, a_dst2, b2):
    ei = edge_index.astype(jnp.int32)
    pad = EPAD - E
    esrc = jnp.concatenate([ei[0], jnp.zeros((pad,), jnp.int32)])
    edst = jnp.concatenate([ei[1], jnp.full((pad,), N, jnp.int32)])
    x_pad = jnp.pad(x, ((0, NP - N), (0, 0)))

    A1 = jnp.concatenate([_expander(a_src1), _expander(a_dst1)], axis=1)
    A2 = jnp.concatenate([_expander(a_src2), _expander(a_dst2)], axis=1)
    Q = _Q
    P = _P
    P16 = _P16
    PM = _PM
    PMT = _PMT
    znum = _ZNUM
    zden = _ZDEN
    b1r = b1.reshape(1, 128)
    b2r = b2.reshape(1, 128)

    h1lo, h1hi, sd1 = _tc1(x_pad, W1, A1, PM)
    h1cat = _pack_rows(h1lo, h1hi)
    num1, den1 = _sc_edge(sd1, h1cat, esrc, edst, znum, zden)
    h2lo, h2hi, sd2 = _tc2(num1, den1, h1lo, h1hi, sd1, b1r, W2, A2,
                           Q, P, P16, PMT, PM)
    h2cat = _pack_rows(h2lo, h2hi)
    num2, den2 = _sc_edge(sd2, h2cat, esrc, edst, znum, zden)
    out = _tc3(num2, den2, h2lo, h2hi, sd2, b2r, Q, P, P16, PMT)
    return out
